# Initial kernel scaffold; baseline (speedup 1.0000x reference)
import jax
import jax.numpy as jnp
from jax.experimental import pallas as pl


def kernel(x, edge_index, Wl1, Wr1, b1, Wl2, Wr2, b2, Wl3, Wr3, b3):
    raise NotImplementedError("write your pallas kernel here")


# speedup vs baseline: 4.3562x; 4.3562x over previous
import functools

import jax
import jax.numpy as jnp
from jax import lax
from jax.experimental import pallas as pl
from jax.experimental.pallas import tpu as pltpu
from jax.experimental.pallas import tpu_sc as plsc

N = 10000
E = 320000
D = 128

NC = 2
NS = 16
NW = NC * NS
EPW = E // NW
K = 80
NCH = EPW // K
RPS = 624
TAIL = N - RPS * NS


def _zero_acc(zeros_hbm, acc_sh, stripe, tail):
    pltpu.sync_copy(zeros_hbm.at[stripe], acc_sh.at[stripe])


def _sc_body(with_cnt, h_hbm, src_hbm, dst_hbm, zeros_hbm, ones_hbm,
             acc_out, cnt_out, src_v, dst_v, rows_v, ones_v, acc_sh, sem):
    c = lax.axis_index("c")
    s = lax.axis_index("s")
    wid = c * NS + s
    stripe = pl.ds(s * RPS, RPS)
    tail = pl.ds(RPS * NS, TAIL)
    base = wid * EPW

    def zero_acc():
        pltpu.sync_copy(zeros_hbm.at[stripe], acc_sh.at[stripe])

        @pl.when(s == 0)
        def _():
            pltpu.sync_copy(zeros_hbm.at[tail], acc_sh.at[tail])

    def write_acc(out):
        pltpu.sync_copy(acc_sh.at[stripe], out.at[c, stripe])

        @pl.when(s == 0)
        def _():
            pltpu.sync_copy(acc_sh.at[tail], out.at[c, tail])

    if with_cnt:
        zero_acc()
        pltpu.sync_copy(ones_hbm, ones_v)
        plsc.subcore_barrier()

        def cchunk(i, carry):
            pltpu.sync_copy(dst_hbm.at[pl.ds(base + i * K, K)], dst_v)
            pltpu.sync_copy(ones_v, acc_sh.at[dst_v], add=True)
            return carry

        lax.fori_loop(0, NCH, cchunk, 0)
        plsc.subcore_barrier()
        write_acc(cnt_out)
        plsc.subcore_barrier()

    zero_acc()
    plsc.subcore_barrier()

    def chunk(i, carry):
        off = base + i * K
        pltpu.sync_copy(src_hbm.at[pl.ds(off, K)], src_v)
        pltpu.sync_copy(dst_hbm.at[pl.ds(off, K)], dst_v)
        pltpu.async_copy(h_hbm.at[src_v], rows_v, sem).wait()
        pltpu.sync_copy(rows_v, acc_sh.at[dst_v], add=True)
        return carry

    lax.fori_loop(0, NCH, chunk, 0)
    plsc.subcore_barrier()
    write_acc(acc_out)


@functools.lru_cache(maxsize=None)
def _make_sc(with_cnt):
    mesh = plsc.VectorSubcoreMesh(core_axis_name="c", subcore_axis_name="s",
                                  num_cores=NC, num_subcores=NS)
    out_type = [jax.ShapeDtypeStruct((NC, N, D), jnp.float32)]
    if with_cnt:
        out_type.append(jax.ShapeDtypeStruct((NC, N, D), jnp.float32))
    scratch = [
        pltpu.VMEM((K,), jnp.int32),
        pltpu.VMEM((K,), jnp.int32),
        pltpu.VMEM((K, D), jnp.float32),
        pltpu.VMEM((K, D), jnp.float32),
        pltpu.VMEM_SHARED((N, D), jnp.float32),
        pltpu.SemaphoreType.DMA,
    ]

    if with_cnt:
        def body(h, src, dst, z, o, acc, cnt, *scr):
            _sc_body(True, h, src, dst, z, o, acc, cnt, *scr)
    else:
        def body(h, src, dst, z, o, acc, *scr):
            _sc_body(False, h, src, dst, z, o, acc, None, *scr)

    return pl.kernel(body, out_type=tuple(out_type), mesh=mesh,
                     scratch_types=scratch)


_R = 1000


def _tc_body(last, acc_ref, cnt_ref, h_ref, wl_ref, wr_ref, b_ref, o_ref):
    agg = acc_ref[0] + acc_ref[1]
    cnt = cnt_ref[0, :, 0:1] + cnt_ref[1, :, 0:1]
    mean = agg / jnp.maximum(cnt, 1.0)
    t = (jnp.dot(mean, wl_ref[...], preferred_element_type=jnp.float32)
         + jnp.dot(h_ref[...], wr_ref[...], preferred_element_type=jnp.float32)
         + b_ref[...])
    sig = jax.nn.sigmoid(t)
    if last:
        @pl.when(pl.program_id(0) == 0)
        def _():
            o_ref[...] = jnp.zeros_like(o_ref)
        o_ref[...] += jnp.sum(sig, axis=0, keepdims=True)
    else:
        o_ref[...] = sig


def _tc_combine(acc, cnt, h, wlT, wrT, b, last):
    grid = (N // _R,)
    in_specs = [
        pl.BlockSpec((NC, _R, D), lambda i: (0, i, 0)),
        pl.BlockSpec((NC, _R, D), lambda i: (0, i, 0)),
        pl.BlockSpec((_R, D), lambda i: (i, 0)),
        pl.BlockSpec((D, D), lambda i: (0, 0)),
        pl.BlockSpec((D, D), lambda i: (0, 0)),
        pl.BlockSpec((1, D), lambda i: (0, 0)),
    ]
    if last:
        out_spec = pl.BlockSpec((1, D), lambda i: (0, 0))
        out_shape = jax.ShapeDtypeStruct((1, D), jnp.float32)
    else:
        out_spec = pl.BlockSpec((_R, D), lambda i: (i, 0))
        out_shape = jax.ShapeDtypeStruct((N, D), jnp.float32)
    return pl.pallas_call(
        functools.partial(_tc_body, last),
        grid=grid,
        in_specs=in_specs,
        out_specs=out_spec,
        out_shape=out_shape,
    )(acc, cnt, h, wlT, wrT, b)


def kernel(x, edge_index, Wl1, Wr1, b1, Wl2, Wr2, b2, Wl3, Wr3, b3):
    src = edge_index[0]
    dst = edge_index[1]
    zeros = jnp.zeros((N, D), jnp.float32)
    ones = jnp.ones((K, D), jnp.float32)

    acc1, cnt = _make_sc(True)(x, src, dst, zeros, ones)
    h2 = _tc_combine(acc1, cnt, x, Wl1.T, Wr1.T, b1.reshape(1, D), False)
    (acc2,) = _make_sc(False)(h2, src, dst, zeros, ones)
    h3 = _tc_combine(acc2, cnt, h2, Wl2.T, Wr2.T, b2.reshape(1, D), False)
    (acc3,) = _make_sc(False)(h3, src, dst, zeros, ones)
    out = _tc_combine(acc3, cnt, h3, Wl3.T, Wr3.T, b3.reshape(1, D), True)
    return out.reshape(D)
